# baseline (speedup 1.0000x reference)
import functools
import math

import jax
import jax.numpy as jnp
from jax import lax
from jax.experimental import pallas as pl
from jax.experimental.pallas import tpu as pltpu
from jax.experimental.pallas import tpu_sc as plsc

_NC, _NS = 2, 16
_NW = _NC * _NS
_BB = 128
_NBUF = 2


@functools.lru_cache(maxsize=None)
def _build(batch, hist, V, D, scale):
    assert batch == _NW * _BB and D % 8 == 0 and hist % _NBUF == 0
    dgroups = D // 8
    mesh = plsc.VectorSubcoreMesh(core_axis_name="c", subcore_axis_name="s")

    @functools.partial(
        pl.kernel,
        out_type=jax.ShapeDtypeStruct((hist, dgroups, _NW, 8, 128),
                                      jnp.float32),
        mesh=mesh,
        compiler_params=pltpu.CompilerParams(use_tc_tiling_on_sc=False,
                                             needs_layout_passes=False),
        scratch_types=[
            pltpu.VMEM((_BB, hist), jnp.int32),
            pltpu.VMEM((hist, _BB), jnp.int32),
            pltpu.VMEM((_BB, D), jnp.float32),
            pltpu.VMEM((_BB, D), jnp.float32),
            pltpu.VMEM((D, _BB), jnp.float32),
            pltpu.VMEM((D, _BB), jnp.float32),
            pltpu.SemaphoreType.DMA,
            pltpu.SemaphoreType.DMA,
            pltpu.SemaphoreType.DMA,
            pltpu.SemaphoreType.DMA,
        ],
    )
    def kern(idx_hbm, table_hbm, out_hbm, idx_v, idxT_v, in0, in1, ot0, ot1,
             g0, g1, s0, s1):
        ins, outs = (in0, in1), (ot0, ot1)
        gsem, ssem = (g0, g1), (s0, s1)
        wid = lax.axis_index("s") * _NC + lax.axis_index("c")
        lane = lax.iota(jnp.int32, 16)

        pltpu.sync_copy(idx_hbm.at[pl.ds(wid * _BB, _BB)], idx_v)

        @plsc.parallel_loop(0, hist)
        def _(t):
            tcol = jnp.full((16,), t, jnp.int32)
            for rq in range(_BB // 16):
                v = plsc.load_gather(idx_v, [rq * 16 + lane, tcol])
                idxT_v[t, pl.ds(rq * 16, 16)] = v

        def gather(t, b):
            return pltpu.make_async_copy(
                table_hbm.at[idxT_v.at[t]], ins[b], gsem[b])

        def scatters(t, b):
            return [
                pltpu.make_async_copy(
                    outs[b].at[pl.ds(dg * 8, 8)],
                    out_hbm.at[t, dg, wid], ssem[b])
                for dg in range(dgroups)
            ]

        for b in range(_NBUF):
            gather(b, b).start()

        @pl.loop(0, hist, step=_NBUF)
        def _(t0):
            for b in range(_NBUF):
                t = t0 + b
                gather(t, b).wait()

                @pl.when(t >= _NBUF)
                def _():
                    for c in scatters(t - _NBUF, b):
                        c.wait()

                @plsc.parallel_loop(0, D)
                def _(d):
                    dsplat = jnp.full((16,), d, jnp.int32)
                    for bq in range(_BB // 16):
                        v = plsc.load_gather(ins[b], [bq * 16 + lane, dsplat])
                        outs[b][d, pl.ds(bq * 16, 16)] = v * scale

                for c in scatters(t, b):
                    c.start()

                @pl.when(t + _NBUF < hist)
                def _():
                    gather(t + _NBUF, b).start()

        for b in range(_NBUF):
            for c in scatters(hist - _NBUF + b, b):
                c.wait()

    return kern


def kernel(input_ids, embeddings):
    batch, hist = input_ids.shape
    V, D = embeddings.shape
    scale = float(math.sqrt(D))
    idx = input_ids.astype(jnp.int32)
    out5 = _build(batch, hist, V, D, scale)(idx, embeddings)
    return out5.transpose(2, 4, 0, 1, 3).reshape(batch, hist, D)


# speedup vs baseline: 1.6319x; 1.6319x over previous
import functools
import math

import jax
import jax.numpy as jnp
from jax import lax
from jax.experimental import pallas as pl
from jax.experimental.pallas import tpu as pltpu
from jax.experimental.pallas import tpu_sc as plsc

_NC, _NS = 2, 16
_NW = _NC * _NS
_BB = 128
_NBUF = 2


@functools.lru_cache(maxsize=None)
def _build(batch, hist, V, D, scale):
    assert batch == _NW * _BB and D % 8 == 0 and hist % _NBUF == 0
    dgroups = D // 8
    mesh = plsc.VectorSubcoreMesh(core_axis_name="c", subcore_axis_name="s")

    @functools.partial(
        pl.kernel,
        out_type=jax.ShapeDtypeStruct((hist, dgroups, _NW, 8, 128),
                                      jnp.float32),
        mesh=mesh,
        compiler_params=pltpu.CompilerParams(use_tc_tiling_on_sc=False,
                                             needs_layout_passes=False),
        scratch_types=[
            pltpu.VMEM((_BB, hist), jnp.int32),
            pltpu.VMEM((hist, _BB), jnp.int32),
            pltpu.VMEM((_BB, D), jnp.float32),
            pltpu.VMEM((_BB, D), jnp.float32),
            pltpu.VMEM((D, _BB), jnp.float32),
            pltpu.VMEM((D, _BB), jnp.float32),
            pltpu.SemaphoreType.DMA,
            pltpu.SemaphoreType.DMA,
            pltpu.SemaphoreType.DMA,
            pltpu.SemaphoreType.DMA,
        ],
    )
    def kern(idx_hbm, table_hbm, out_hbm, idx_v, idxT_v, in0, in1, ot0, ot1,
             g0, g1, s0, s1):
        ins, outs = (in0, in1), (ot0, ot1)
        gsem, ssem = (g0, g1), (s0, s1)
        wid = lax.axis_index("s") * _NC + lax.axis_index("c")
        lane = lax.iota(jnp.int32, 16)

        pltpu.sync_copy(idx_hbm.at[pl.ds(wid * _BB, _BB)], idx_v)

        toffs = list(range(0, hist - 15, 16))
        if toffs[-1] + 16 < hist:
            toffs.append(hist - 16)

        @plsc.parallel_loop(0, _BB)
        def _(r):
            rcol = jnp.full((16,), r, jnp.int32)
            for to in toffs:
                v = idx_v[r, pl.ds(to, 16)]
                plsc.store_scatter(idxT_v, [to + lane, rcol], v)

        def gather(t, b):
            return pltpu.make_async_copy(
                table_hbm.at[idxT_v.at[t]], ins[b], gsem[b])

        def scatters(t, b):
            return [
                pltpu.make_async_copy(
                    outs[b].at[pl.ds(dg * 8, 8)],
                    out_hbm.at[t, dg, wid], ssem[b])
                for dg in range(dgroups)
            ]

        for b in range(_NBUF):
            gather(b, b).start()

        @pl.loop(0, hist, step=_NBUF)
        def _(t0):
            for b in range(_NBUF):
                t = t0 + b
                gather(t, b).wait()

                @pl.when(t >= _NBUF)
                def _():
                    for c in scatters(t - _NBUF, b):
                        c.wait()

                @functools.partial(plsc.parallel_loop, 0, _BB, unroll=2)
                def _(r):
                    rcol = jnp.full((16,), r, jnp.int32)
                    for dq in range(D // 16):
                        v = ins[b][r, pl.ds(dq * 16, 16)] * scale
                        plsc.store_scatter(outs[b], [dq * 16 + lane, rcol], v)

                for c in scatters(t, b):
                    c.start()

                @pl.when(t + _NBUF < hist)
                def _():
                    gather(t + _NBUF, b).start()

        for b in range(_NBUF):
            for c in scatters(hist - _NBUF + b, b):
                c.wait()

    return kern


def kernel(input_ids, embeddings):
    batch, hist = input_ids.shape
    V, D = embeddings.shape
    scale = float(math.sqrt(D))
    idx = input_ids.astype(jnp.int32)
    emb = lax.optimization_barrier(embeddings.reshape(-1)).reshape(V, D)
    out5 = _build(batch, hist, V, D, scale)(idx, emb)
    return out5.transpose(2, 4, 0, 1, 3).reshape(batch, hist, D)
